# baseline (speedup 1.0000x reference)
import functools

import jax
import jax.numpy as jnp
from jax import lax
from jax.experimental import pallas as pl
from jax.experimental.pallas import tpu as pltpu
from jax.experimental.pallas import tpu_sc as plsc


def _sc_call(b, d, h, w, row_embed, col_embed):
    mesh = plsc.VectorSubcoreMesh(core_axis_name="c", subcore_axis_name="s")
    n_workers = 32
    y_groups = n_workers // b
    rows_per_w = h // y_groups

    @functools.partial(
        pl.kernel,
        out_type=jax.ShapeDtypeStruct((b, h, w, 2 * d), jnp.float32),
        mesh=mesh,
        scratch_types=[
            pltpu.VMEM((w, d), jnp.float32),
            pltpu.VMEM((rows_per_w, d), jnp.float32),
            pltpu.SemaphoreType.DMA,
        ],
    )
    def sck(row_hbm, col_hbm, out_hbm, cbuf, rbuf, sem):
        wid = lax.axis_index("s") * 2 + lax.axis_index("c")
        b_i = wid // y_groups
        y0 = (wid % y_groups) * rows_per_w
        pltpu.sync_copy(col_hbm.at[pl.ds(0, w), :], cbuf)
        pltpu.sync_copy(row_hbm.at[pl.ds(y0, rows_per_w), :], rbuf)
        copies = []
        for yi in range(rows_per_w):
            y = y0 + yi
            copies.append(pltpu.make_async_copy(
                cbuf, out_hbm.at[b_i, y, :, pl.ds(0, d)], sem))
            for x in range(w):
                copies.append(pltpu.make_async_copy(
                    rbuf.at[yi], out_hbm.at[b_i, y, x, pl.ds(d, d)], sem))
        for c in copies:
            c.start()
        for c in copies:
            c.wait()

    return sck(row_embed, col_embed)


def kernel(tensor, row_embed, col_embed):
    b = tensor.shape[0]
    h, w = tensor.shape[-2], tensor.shape[-1]
    d = row_embed.shape[1]

    out = _sc_call(b, d, h, w, row_embed, col_embed)
    return out.transpose(0, 3, 1, 2)
